# baseline (speedup 1.0000x reference)
import functools

import jax
import jax.numpy as jnp
from jax import lax
from jax.experimental import pallas as pl
from jax.experimental.pallas import tpu as pltpu
from jax.experimental.pallas import tpu_sc as plsc

_NC, _NS = 2, 16
_NW = _NC * _NS
_BSZ, _SEQ = 4, 4096
_B = _BSZ * _SEQ
_BW = _B // _NW
_TPB = _SEQ // _BW
_D = 128
_C = 128
_NCHUNK = _BW // _C
_NSLOT = 4
_LAG = 2

_mesh = plsc.VectorSubcoreMesh(core_axis_name="c", subcore_axis_name="s")


@functools.partial(
    pl.kernel,
    out_type=(
        jax.ShapeDtypeStruct((_BSZ, _SEQ, _D), jnp.float32),
        jax.ShapeDtypeStruct((_BSZ, _SEQ, _D), jnp.float32),
    ),
    mesh=_mesh,
    scratch_types=[
        pltpu.VMEM((_NCHUNK, _C), jnp.int32),
        [pltpu.VMEM((_C, _D), jnp.float32) for _ in range(_NSLOT)],
        [pltpu.SemaphoreType.DMA for _ in range(_NSLOT)],
        [pltpu.SemaphoreType.DMA for _ in range(_NSLOT)],
    ],
)
def _rope_gather(cos_hbm, sin_hbm, idx_hbm, cos_out, sin_out,
                 idx_v, bufs, gsems, wsems):
    wid = lax.axis_index("s") * _NC + lax.axis_index("c")
    b = wid // _TPB
    s0 = (wid % _TPB) * _BW
    pltpu.sync_copy(idx_hbm.at[wid], idx_v)

    njobs = 2 * _NCHUNK
    gathers = [None] * njobs
    writes = [None] * njobs

    def job(j):
        tbl, out = (cos_hbm, cos_out) if j < _NCHUNK else (sin_hbm, sin_out)
        c = j % _NCHUNK
        return tbl, out, c

    for step in range(njobs + _LAG):
        if step < njobs:
            s = step % _NSLOT
            tbl, _, c = job(step)
            if step >= _NSLOT:
                writes[step - _NSLOT].wait()
            gathers[step] = pltpu.async_copy(
                tbl.at[idx_v.at[c]], bufs[s], gsems[s])
        k = step - _LAG
        if 0 <= k < njobs:
            s = k % _NSLOT
            _, out, c = job(k)
            gathers[k].wait()
            writes[k] = pltpu.async_copy(
                bufs[s], out.at[b, pl.ds(s0 + c * _C, _C), :], wsems[s])
    for k in range(njobs - _NSLOT, njobs):
        writes[k].wait()


def kernel(x, position_ids, cos_cached, sin_cached):
    idx = position_ids.reshape(_NW, _NCHUNK, _C).astype(jnp.int32)
    cos, sin = _rope_gather(cos_cached, sin_cached, idx)
    return (cos.astype(x.dtype), sin.astype(x.dtype))


# speedup vs baseline: 1.0282x; 1.0282x over previous
import functools

import jax
import jax.numpy as jnp
from jax import lax
from jax.experimental import pallas as pl
from jax.experimental.pallas import tpu as pltpu
from jax.experimental.pallas import tpu_sc as plsc

_NC, _NS = 2, 16
_NW = _NC * _NS
_BSZ, _SEQ = 4, 4096
_B = _BSZ * _SEQ
_BW = _B // _NW
_TPB = _SEQ // _BW
_D = 128

_mesh = plsc.VectorSubcoreMesh(core_axis_name="c", subcore_axis_name="s")


@functools.partial(
    pl.kernel,
    out_type=(
        jax.ShapeDtypeStruct((_BSZ, _SEQ, _D), jnp.float32),
        jax.ShapeDtypeStruct((_BSZ, _SEQ, _D), jnp.float32),
    ),
    mesh=_mesh,
    scratch_types=[
        pltpu.VMEM((_BW,), jnp.int32),
        pltpu.VMEM((_BW, _D), jnp.float32),
        pltpu.SemaphoreType.DMA,
    ],
)
def _rope_gather(cos_hbm, sin_hbm, idx_hbm, cos_out, sin_out,
                 idx_v, rows_v, sem):
    wid = lax.axis_index("s") * _NC + lax.axis_index("c")
    b = wid // _TPB
    s0 = (wid % _TPB) * _BW
    rows = pl.ds(s0, _BW)
    pltpu.sync_copy(idx_hbm.at[b, rows], idx_v)
    pltpu.async_copy(cos_hbm.at[idx_v], rows_v, sem).wait()
    pltpu.sync_copy(rows_v, cos_out.at[b, rows, :])
    pltpu.async_copy(sin_hbm.at[idx_v], rows_v, sem).wait()
    pltpu.sync_copy(rows_v, sin_out.at[b, rows, :])


def kernel(x, position_ids, cos_cached, sin_cached):
    idx = position_ids.astype(jnp.int32)
    cos, sin = _rope_gather(cos_cached, sin_cached, idx)
    return (cos.astype(x.dtype), sin.astype(x.dtype))
